# Initial kernel scaffold; baseline (speedup 1.0000x reference)
import jax
import jax.numpy as jnp
from jax.experimental import pallas as pl


def kernel(pts3D, src):
    raise NotImplementedError("write your pallas kernel here")


# speedup vs baseline: 275.0647x; 275.0647x over previous
import functools

import jax
import jax.numpy as jnp
from jax import lax
from jax.experimental import pallas as pl
from jax.experimental.pallas import tpu as pltpu
from jax.experimental.pallas import tpu_sc as plsc

B = 2
P = 4096
C = 64
S = 256
K = 8
RADIUS = 1.5 / S * 2.0
R2 = RADIUS * RADIUS
INV_R2 = 1.0 / R2

NW = 32
ROWS = B * S
ROWS_PER_W = ROWS // NW
NV = S // 16
PV = P // 16
ENT_CAP = K * S + 16
INF = float("inf")


def _sqrt16(v):
    u = plsc.bitcast(v, jnp.int32)
    u = jnp.int32(0x5F3759DF) - lax.shift_right_logical(u, 1)
    y = plsc.bitcast(u, jnp.float32)
    for _ in range(3):
        y = y * (1.5 - 0.5 * v * y * y)
    return v * y


def _raster_body(xs_h, ys_h, zs_h, feats_h, out_h, empty_h,
                 xv, yv, zv, pxv, slz, sld, sli,
                 candp, candx, candz, candd,
                 ent_i, ent_w, ent_p, emptyv, fbuf, idxq, orow, sem):
    cid = lax.axis_index("c")
    sid = lax.axis_index("s")
    wid = sid * 2 + cid

    pltpu.sync_copy(xs_h, xv)
    pltpu.sync_copy(ys_h, yv)
    pltpu.sync_copy(zs_h, zv)

    iota = lax.iota(jnp.int32, 16)
    iotaf = iota.astype(jnp.float32)

    def px_init(g, _):
        colf = iotaf + jnp.float32(g * 16)
        pxv[pl.ds(g * 16, 16)] = 1.0 - (2.0 * colf + 1.0) * jnp.float32(1.0 / S)
        return 0
    lax.fori_loop(0, NV, px_init, 0)

    zeros16 = jnp.zeros((16,), jnp.float32)
    zeros16i = jnp.zeros((16,), jnp.int32)
    inf16 = jnp.full((16,), INF, jnp.float32)

    def row_body(t, _):
        g_row = wid + NW * t
        b = g_row // S
        j = g_row - b * S
        pyj = 1.0 - (2.0 * jnp.float32(j) + 1.0) * jnp.float32(1.0 / S)
        pbase = b * P

        def zslots(i, _):
            slz[pl.ds(i * 16, 16)] = inf16
            return 0
        lax.fori_loop(0, K * NV, zslots, 0)

        def zorow(i, _):
            orow[pl.ds(i * 16, 16)] = zeros16
            return 0
        lax.fori_loop(0, S * C // 16, zorow, 0)

        def scan_body(i, cnt):
            off = pbase + i * 16
            vy = yv[pl.ds(off, 16)]
            dy = pyj + vy
            dy2 = dy * dy
            m = dy2 < R2
            n = plsc.all_reduce_population_count(m)[0]
            pid = iota + off
            plsc.store_compressed(candp.at[pl.ds(cnt, 16)], pid, mask=m)
            plsc.store_compressed(candx.at[pl.ds(cnt, 16)], xv[pl.ds(off, 16)], mask=m)
            plsc.store_compressed(candz.at[pl.ds(cnt, 16)], zv[pl.ds(off, 16)], mask=m)
            plsc.store_compressed(candd.at[pl.ds(cnt, 16)], dy2, mask=m)
            return cnt + n
        cnt = lax.fori_loop(0, PV, scan_body, jnp.int32(0))

        def cand_body(ci, _):
            x = candx[pl.ds(ci, 16)][0]
            z = candz[pl.ds(ci, 16)][0]
            dy2 = candd[pl.ds(ci, 16)][0]
            pid = candp[pl.ds(ci, 16)][0]
            lo_f = jnp.clip((1.0 + x - RADIUS) * (S * 0.5) - 1.5, 0.0, S - 1.0)
            hi_f = jnp.clip((1.0 + x + RADIUS) * (S * 0.5) + 0.5, 0.0, S - 1.0)
            i_lo = lo_f.astype(jnp.int32)
            i_hi = hi_f.astype(jnp.int32)
            v0 = lax.shift_right_logical(i_lo, 4)
            v1 = lax.shift_right_logical(i_hi, 4)

            def group(g):
                base = g * 16
                pxg = pxv[pl.ds(base, 16)]
                dx = pxg + x
                d2 = dx * dx + dy2
                m = d2 < R2
                cz = jnp.where(m, z, INF)
                cd = d2
                cidx = jnp.full((16,), pid, jnp.int32)
                for s in range(K):
                    off = s * S + base
                    sz = slz[pl.ds(off, 16)]
                    sd = sld[pl.ds(off, 16)]
                    si = sli[pl.ds(off, 16)]
                    take = cz < sz
                    slz[pl.ds(off, 16)] = jnp.where(take, cz, sz)
                    sld[pl.ds(off, 16)] = jnp.where(take, cd, sd)
                    sli[pl.ds(off, 16)] = jnp.where(take, cidx, si)
                    cz = jnp.where(take, sz, cz)
                    cd = jnp.where(take, sd, cd)
                    cidx = jnp.where(take, si, cidx)

            group(v0)

            @pl.when(v1 > v0)
            def _():
                group(v1)
            return 0
        lax.fori_loop(0, cnt, cand_body, 0)

        def comp_body(g, ec):
            base = g * 16
            pxcol = iota + base
            tvec = jnp.full((16,), jnp.float32(1.0))
            for s in range(K):
                off = s * S + base
                zs_ = slz[pl.ds(off, 16)]
                d2s = sld[pl.ds(off, 16)]
                pis = sli[pl.ds(off, 16)]
                valid = zs_ < INF
                if s == 0:
                    emptyv[pl.ds(base, 16)] = jnp.where(valid, 0.0, 1.0)
                dist = jnp.clip(d2s * INV_R2, 0.001, 1.0)
                a = jnp.maximum(1.0 - _sqrt16(dist), 0.0)
                a = jnp.where(valid, a, 0.0)
                w = a * tvec
                tvec = tvec * (1.0 - a)
                m = w > 0.0
                n = plsc.all_reduce_population_count(m)[0]
                plsc.store_compressed(ent_i.at[pl.ds(ec, 16)], pis, mask=m)
                plsc.store_compressed(ent_w.at[pl.ds(ec, 16)], w, mask=m)
                plsc.store_compressed(ent_p.at[pl.ds(ec, 16)], pxcol, mask=m)
                ec = ec + n
            return ec
        ec = lax.fori_loop(0, NV, comp_body, jnp.int32(0))

        ent_i[pl.ds(ec, 16)] = zeros16i
        ent_w[pl.ds(ec, 16)] = zeros16
        ent_p[pl.ds(ec, 16)] = zeros16i

        nch = lax.shift_right_logical(ec + 15, 4)

        def chunk_body(ch, _):
            idxq[...] = ent_i[pl.ds(ch * 16, 16)]
            pltpu.async_copy(feats_h.at[idxq], fbuf, sem).wait()
            wv = ent_w[pl.ds(ch * 16, 16)]
            pv = ent_p[pl.ds(ch * 16, 16)]
            for e in range(16):
                w_e = wv[e]
                px_e = pv[e]
                obase = px_e * C
                for q in range(C // 16):
                    plsc.addupdate(orow.at[pl.ds(obase + q * 16, 16)],
                                   w_e * fbuf[e, pl.ds(q * 16, 16)])
            return 0
        lax.fori_loop(0, nch, chunk_body, 0)

        pltpu.sync_copy(orow, out_h.at[g_row])
        pltpu.sync_copy(emptyv, empty_h.at[g_row])
        return 0

    lax.fori_loop(0, ROWS_PER_W, row_body, 0)


def _sc_raster(xs, ys, zs, feats):
    mesh = plsc.VectorSubcoreMesh(core_axis_name="c", subcore_axis_name="s")
    f = pl.kernel(
        _raster_body,
        out_type=(
            jax.ShapeDtypeStruct((ROWS, S * C), jnp.float32),
            jax.ShapeDtypeStruct((ROWS, S), jnp.float32),
        ),
        mesh=mesh,
        compiler_params=pltpu.CompilerParams(needs_layout_passes=False,
                                             use_tc_tiling_on_sc=False),
        scratch_types=[
            pltpu.VMEM((B * P,), jnp.float32),
            pltpu.VMEM((B * P,), jnp.float32),
            pltpu.VMEM((B * P,), jnp.float32),
            pltpu.VMEM((S,), jnp.float32),
            pltpu.VMEM((K * S,), jnp.float32),
            pltpu.VMEM((K * S,), jnp.float32),
            pltpu.VMEM((K * S,), jnp.int32),
            pltpu.VMEM((P + 16,), jnp.int32),
            pltpu.VMEM((P + 16,), jnp.float32),
            pltpu.VMEM((P + 16,), jnp.float32),
            pltpu.VMEM((P + 16,), jnp.float32),
            pltpu.VMEM((ENT_CAP,), jnp.int32),
            pltpu.VMEM((ENT_CAP,), jnp.float32),
            pltpu.VMEM((ENT_CAP,), jnp.int32),
            pltpu.VMEM((S,), jnp.float32),
            pltpu.VMEM((16, C), jnp.float32),
            pltpu.VMEM((16,), jnp.int32),
            pltpu.VMEM((S * C,), jnp.float32),
            pltpu.SemaphoreType.DMA,
        ],
    )
    return f(xs, ys, zs, feats)


def _bg_body(e_ref, m_ref, o_ref):
    e = e_ref[0]
    m = m_ref[...]
    t = jax.lax.dot_general(m, e, (((1,), (0,)), ((), ())),
                            preferred_element_type=jnp.float32)
    t2 = jax.lax.dot_general(t, m, (((1,), (0,)), ((), ())),
                             preferred_element_type=jnp.float32)
    o_ref[0] = (t2 > 0.0).astype(jnp.float32)


def _bg_dilate(empty):
    band = (jnp.abs(jnp.arange(S)[:, None] - jnp.arange(S)[None, :]) <= 1)
    m = band.astype(jnp.float32)
    return pl.pallas_call(
        _bg_body,
        out_shape=jax.ShapeDtypeStruct((B, S, S), jnp.float32),
        grid=(B,),
        in_specs=[pl.BlockSpec((1, S, S), lambda i: (i, 0, 0)),
                  pl.BlockSpec((S, S), lambda i: (0, 0))],
        out_specs=pl.BlockSpec((1, S, S), lambda i: (i, 0, 0)),
    )(empty, m)


def kernel(pts3D, src):
    xs = pts3D[..., 0].reshape(B * P)
    ys = pts3D[..., 1].reshape(B * P)
    zs = pts3D[..., 2].reshape(B * P)
    feats = jnp.transpose(src, (0, 2, 1)).reshape(B * P, C)
    out_hwc, empty = _sc_raster(xs, ys, zs, feats)
    out = jnp.transpose(out_hwc.reshape(B, S, S, C), (0, 3, 1, 2))
    bg = _bg_dilate(empty.reshape(B, S, S))
    return out, bg.astype(jnp.bool_)


# speedup vs baseline: 307.2443x; 1.1170x over previous
import functools

import jax
import jax.numpy as jnp
from jax import lax
from jax.experimental import pallas as pl
from jax.experimental.pallas import tpu as pltpu
from jax.experimental.pallas import tpu_sc as plsc

B = 2
P = 4096
C = 64
S = 256
K = 8
RADIUS = 1.5 / S * 2.0
R2 = RADIUS * RADIUS
INV_R2 = 1.0 / R2

NW = 32
ROWS = B * S
ROWS_PER_W = ROWS // NW
NV = S // 16
PV = P // 16
ENT_CAP = K * S + 16
INF = float("inf")


def _sqrt16(v):
    u = plsc.bitcast(v, jnp.int32)
    u = jnp.int32(0x5F3759DF) - lax.shift_right_logical(u, 1)
    y = plsc.bitcast(u, jnp.float32)
    for _ in range(3):
        y = y * (1.5 - 0.5 * v * y * y)
    return v * y


def _raster_body(xs_h, ys_h, zs_h, feats_h, out_h, empty_h,
                 xv, yv, zv, pxv, slz, sld, sli,
                 candp, candx, candz, candd,
                 ent_i, ent_w, ent_p, emptyv, fbuf, idxq, orow, occs, sem):
    cid = lax.axis_index("c")
    sid = lax.axis_index("s")
    wid = sid * 2 + cid

    pltpu.sync_copy(xs_h, xv)
    pltpu.sync_copy(ys_h, yv)
    pltpu.sync_copy(zs_h, zv)

    iota = lax.iota(jnp.int32, 16)
    iotaf = iota.astype(jnp.float32)

    def px_init(g, _):
        colf = iotaf + jnp.float32(g * 16)
        pxv[pl.ds(g * 16, 16)] = 1.0 - (2.0 * colf + 1.0) * jnp.float32(1.0 / S)
        return 0
    lax.fori_loop(0, NV, px_init, 0)

    zeros16 = jnp.zeros((16,), jnp.float32)
    zeros16i = jnp.zeros((16,), jnp.int32)
    inf16 = jnp.full((16,), INF, jnp.float32)

    def row_body(t, _):
        g_row = wid + NW * t
        b = g_row // S
        j = g_row - b * S
        pyj = 1.0 - (2.0 * jnp.float32(j) + 1.0) * jnp.float32(1.0 / S)
        pbase = b * P

        def zslots(i, _):
            for u in range(4):
                slz[pl.ds(i * 64 + u * 16, 16)] = inf16
            return 0
        lax.fori_loop(0, K * NV // 4, zslots, 0)

        def zorow(i, _):
            for u in range(4):
                orow[pl.ds(i * 64 + u * 16, 16)] = zeros16
            return 0
        lax.fori_loop(0, S * C // 64, zorow, 0)

        def zocc(i, _):
            occs[i] = jnp.int32(0)
            return 0
        lax.fori_loop(0, NV, zocc, 0)

        def scan_body(i, cnt):
            off = pbase + i * 16
            vy = yv[pl.ds(off, 16)]
            dy = pyj + vy
            dy2 = dy * dy
            m = dy2 < R2
            n = plsc.all_reduce_population_count(m)[0]

            @pl.when(n > 0)
            def _():
                pid = iota + off
                plsc.store_compressed(candp.at[pl.ds(cnt, 16)], pid, mask=m)
                plsc.store_compressed(candx.at[pl.ds(cnt, 16)], xv[pl.ds(off, 16)], mask=m)
                plsc.store_compressed(candz.at[pl.ds(cnt, 16)], zv[pl.ds(off, 16)], mask=m)
                plsc.store_compressed(candd.at[pl.ds(cnt, 16)], dy2, mask=m)
            return cnt + n
        cnt = lax.fori_loop(0, PV, scan_body, jnp.int32(0))

        def cand_body(ci, _):
            x = candx[pl.ds(ci, 16)][0]
            z = candz[pl.ds(ci, 16)][0]
            dy2 = candd[pl.ds(ci, 16)][0]
            pid = candp[pl.ds(ci, 16)][0]
            lo_f = jnp.clip((1.0 + x - RADIUS) * (S * 0.5) - 1.5, 0.0, S - 1.0)
            hi_f = jnp.clip((1.0 + x + RADIUS) * (S * 0.5) + 0.5, 0.0, S - 1.0)
            i_lo = lo_f.astype(jnp.int32)
            i_hi = hi_f.astype(jnp.int32)
            v0 = lax.shift_right_logical(i_lo, 4)
            v1 = lax.shift_right_logical(i_hi, 4)

            def group(g):
                base = g * 16
                pxg = pxv[pl.ds(base, 16)]
                dx = pxg + x
                d2 = dx * dx + dy2
                m = d2 < R2
                nm = plsc.all_reduce_population_count(m)[0]

                @pl.when(nm > 0)
                def _():
                    cz0 = jnp.where(m, z, INF)
                    cidx0 = jnp.full((16,), pid, jnp.int32)
                    og = occs[g]
                    nit = jnp.minimum(og + 1, K)

                    def ins(s, carry):
                        cz, cd, cidx = carry
                        off = s * S + base
                        sz = slz[pl.ds(off, 16)]
                        sd = sld[pl.ds(off, 16)]
                        si = sli[pl.ds(off, 16)]
                        take = cz < sz
                        slz[pl.ds(off, 16)] = jnp.where(take, cz, sz)
                        sld[pl.ds(off, 16)] = jnp.where(take, cd, sd)
                        sli[pl.ds(off, 16)] = jnp.where(take, cidx, si)
                        return (jnp.where(take, sz, cz),
                                jnp.where(take, sd, cd),
                                jnp.where(take, si, cidx))
                    lax.fori_loop(0, nit, ins, (cz0, d2, cidx0))
                    occs[g] = nit

            group(v0)

            @pl.when(v1 > v0)
            def _():
                group(v1)
            return 0
        lax.fori_loop(0, cnt, cand_body, 0)

        def comp_body(g, ec):
            base = g * 16
            pxcol = iota + base
            emptyv[pl.ds(base, 16)] = jnp.where(slz[pl.ds(base, 16)] < INF,
                                                0.0, 1.0)
            og = occs[g]

            def slot(s, carry):
                ec, tvec = carry
                off = s * S + base
                zs_ = slz[pl.ds(off, 16)]
                d2s = sld[pl.ds(off, 16)]
                pis = sli[pl.ds(off, 16)]
                valid = zs_ < INF
                dist = jnp.clip(d2s * INV_R2, 0.001, 1.0)
                a = jnp.maximum(1.0 - _sqrt16(dist), 0.0)
                a = jnp.where(valid, a, 0.0)
                w = a * tvec
                tvec = tvec * (1.0 - a)
                m = w > 0.0
                n = plsc.all_reduce_population_count(m)[0]
                plsc.store_compressed(ent_i.at[pl.ds(ec, 16)], pis, mask=m)
                plsc.store_compressed(ent_w.at[pl.ds(ec, 16)], w, mask=m)
                plsc.store_compressed(ent_p.at[pl.ds(ec, 16)], pxcol, mask=m)
                return (ec + n, tvec)
            ec, _unused = lax.fori_loop(
                0, og, slot, (ec, jnp.full((16,), jnp.float32(1.0))))
            return ec
        ec = lax.fori_loop(0, NV, comp_body, jnp.int32(0))

        ent_i[pl.ds(ec, 16)] = zeros16i
        ent_w[pl.ds(ec, 16)] = zeros16
        ent_p[pl.ds(ec, 16)] = zeros16i

        nch = lax.shift_right_logical(ec + 15, 4)

        def chunk_body(ch, _):
            idxq[...] = ent_i[pl.ds(ch * 16, 16)]
            pltpu.async_copy(feats_h.at[idxq], fbuf, sem).wait()
            wv = ent_w[pl.ds(ch * 16, 16)]
            pv = ent_p[pl.ds(ch * 16, 16)]
            for e in range(16):
                w_e = wv[e]
                px_e = pv[e]
                obase = px_e * C
                for q in range(C // 16):
                    plsc.addupdate(orow.at[pl.ds(obase + q * 16, 16)],
                                   w_e * fbuf[e, pl.ds(q * 16, 16)])
            return 0
        lax.fori_loop(0, nch, chunk_body, 0)

        pltpu.sync_copy(orow, out_h.at[g_row])
        pltpu.sync_copy(emptyv, empty_h.at[g_row])
        return 0

    lax.fori_loop(0, ROWS_PER_W, row_body, 0)


def _sc_raster(xs, ys, zs, feats):
    mesh = plsc.VectorSubcoreMesh(core_axis_name="c", subcore_axis_name="s")
    f = pl.kernel(
        _raster_body,
        out_type=(
            jax.ShapeDtypeStruct((ROWS, S * C), jnp.float32),
            jax.ShapeDtypeStruct((ROWS, S), jnp.float32),
        ),
        mesh=mesh,
        compiler_params=pltpu.CompilerParams(needs_layout_passes=False,
                                             use_tc_tiling_on_sc=False),
        scratch_types=[
            pltpu.VMEM((B * P,), jnp.float32),
            pltpu.VMEM((B * P,), jnp.float32),
            pltpu.VMEM((B * P,), jnp.float32),
            pltpu.VMEM((S,), jnp.float32),
            pltpu.VMEM((K * S,), jnp.float32),
            pltpu.VMEM((K * S,), jnp.float32),
            pltpu.VMEM((K * S,), jnp.int32),
            pltpu.VMEM((P + 16,), jnp.int32),
            pltpu.VMEM((P + 16,), jnp.float32),
            pltpu.VMEM((P + 16,), jnp.float32),
            pltpu.VMEM((P + 16,), jnp.float32),
            pltpu.VMEM((ENT_CAP,), jnp.int32),
            pltpu.VMEM((ENT_CAP,), jnp.float32),
            pltpu.VMEM((ENT_CAP,), jnp.int32),
            pltpu.VMEM((S,), jnp.float32),
            pltpu.VMEM((16, C), jnp.float32),
            pltpu.VMEM((16,), jnp.int32),
            pltpu.VMEM((S * C,), jnp.float32),
            pltpu.SMEM((NV,), jnp.int32),
            pltpu.SemaphoreType.DMA,
        ],
    )
    return f(xs, ys, zs, feats)


def _bg_body(e_ref, m_ref, o_ref):
    e = e_ref[0]
    m = m_ref[...]
    t = jax.lax.dot_general(m, e, (((1,), (0,)), ((), ())),
                            preferred_element_type=jnp.float32)
    t2 = jax.lax.dot_general(t, m, (((1,), (0,)), ((), ())),
                             preferred_element_type=jnp.float32)
    o_ref[0] = (t2 > 0.0).astype(jnp.float32)


def _bg_dilate(empty):
    band = (jnp.abs(jnp.arange(S)[:, None] - jnp.arange(S)[None, :]) <= 1)
    m = band.astype(jnp.float32)
    return pl.pallas_call(
        _bg_body,
        out_shape=jax.ShapeDtypeStruct((B, S, S), jnp.float32),
        grid=(B,),
        in_specs=[pl.BlockSpec((1, S, S), lambda i: (i, 0, 0)),
                  pl.BlockSpec((S, S), lambda i: (0, 0))],
        out_specs=pl.BlockSpec((1, S, S), lambda i: (i, 0, 0)),
    )(empty, m)


def kernel(pts3D, src):
    xs = pts3D[..., 0].reshape(B * P)
    ys = pts3D[..., 1].reshape(B * P)
    zs = pts3D[..., 2].reshape(B * P)
    feats = jnp.transpose(src, (0, 2, 1)).reshape(B * P, C)
    out_hwc, empty = _sc_raster(xs, ys, zs, feats)
    out = jnp.transpose(out_hwc.reshape(B, S, S, C), (0, 3, 1, 2))
    bg = _bg_dilate(empty.reshape(B, S, S))
    return out, bg.astype(jnp.bool_)


# speedup vs baseline: 326.2224x; 1.0618x over previous
import functools

import jax
import jax.numpy as jnp
from jax import lax
from jax.experimental import pallas as pl
from jax.experimental.pallas import tpu as pltpu
from jax.experimental.pallas import tpu_sc as plsc

B = 2
P = 4096
C = 64
S = 256
K = 8
RADIUS = 1.5 / S * 2.0
R2 = RADIUS * RADIUS
INV_R2 = 1.0 / R2

NW = 32
ROWS = B * S
ROWS_PER_W = ROWS // NW
NV = S // 16
PV = P // 16
ENT_CAP = K * S + 16
NWIN = 48
INF = float("inf")


def _sqrt16(v):
    u = plsc.bitcast(v, jnp.int32)
    u = jnp.int32(0x5F3759DF) - lax.shift_right_logical(u, 1)
    y = plsc.bitcast(u, jnp.float32)
    for _ in range(3):
        y = y * (1.5 - 0.5 * v * y * y)
    return v * y


def _raster_body(xs_h, ys_h, zs_h, feats_h, out_h, empty_h,
                 xv, yv, zv, pxv, slz, sld, sli,
                 candp, candx, candz, candd,
                 ent_i, ent_w, ent_p, emptyv, fbuf, orow, occs, sem):
    cid = lax.axis_index("c")
    sid = lax.axis_index("s")
    wid = sid * 2 + cid

    pltpu.sync_copy(xs_h, xv)
    pltpu.sync_copy(ys_h, yv)
    pltpu.sync_copy(zs_h, zv)

    iota = lax.iota(jnp.int32, 16)
    iotaf = iota.astype(jnp.float32)

    def px_init(g, _):
        colf = iotaf + jnp.float32(g * 16)
        pxv[pl.ds(g * 16, 16)] = 1.0 - (2.0 * colf + 1.0) * jnp.float32(1.0 / S)
        return 0
    lax.fori_loop(0, NV, px_init, 0)

    zeros16 = jnp.zeros((16,), jnp.float32)
    zeros16i = jnp.zeros((16,), jnp.int32)
    inf16 = jnp.full((16,), INF, jnp.float32)

    def row_body(t, _):
        g_row = wid + NW * t
        b = g_row // S
        j = g_row - b * S
        pyj = 1.0 - (2.0 * jnp.float32(j) + 1.0) * jnp.float32(1.0 / S)
        pbase = b * P

        def zslots(i, _):
            for u in range(4):
                slz[pl.ds(i * 64 + u * 16, 16)] = inf16
            return 0
        lax.fori_loop(0, K * NV // 4, zslots, 0)

        def zorow(i, _):
            for u in range(4):
                orow[pl.ds(i * 64 + u * 16, 16)] = zeros16
            return 0
        lax.fori_loop(0, S * C // 64, zorow, 0)

        def zocc(i, _):
            occs[i] = jnp.int32(0)
            return 0
        lax.fori_loop(0, NV, zocc, 0)

        def scan_body(i, cnt):
            off = pbase + i * 16
            vy = yv[pl.ds(off, 16)]
            dy = pyj + vy
            dy2 = dy * dy
            m = dy2 < R2
            n = plsc.all_reduce_population_count(m)[0]

            @pl.when(n > 0)
            def _():
                pid = iota + off
                plsc.store_compressed(candp.at[pl.ds(cnt, 16)], pid, mask=m)
                plsc.store_compressed(candx.at[pl.ds(cnt, 16)], xv[pl.ds(off, 16)], mask=m)
                plsc.store_compressed(candz.at[pl.ds(cnt, 16)], zv[pl.ds(off, 16)], mask=m)
                plsc.store_compressed(candd.at[pl.ds(cnt, 16)], dy2, mask=m)
            return cnt + n
        cnt = lax.fori_loop(0, PV, scan_body, jnp.int32(0))

        def cand_body(ci, _):
            x = candx[pl.ds(ci, 16)][0]
            z = candz[pl.ds(ci, 16)][0]
            dy2 = candd[pl.ds(ci, 16)][0]
            pid = candp[pl.ds(ci, 16)][0]
            lo_f = jnp.clip((1.0 + x - RADIUS) * (S * 0.5) - 1.5, 0.0, S - 1.0)
            hi_f = jnp.clip((1.0 + x + RADIUS) * (S * 0.5) + 0.5, 0.0, S - 1.0)
            i_lo = lo_f.astype(jnp.int32)
            i_hi = hi_f.astype(jnp.int32)
            v0 = lax.shift_right_logical(i_lo, 4)
            v1 = lax.shift_right_logical(i_hi, 4)

            def group(g):
                base = g * 16
                pxg = pxv[pl.ds(base, 16)]
                dx = pxg + x
                d2 = dx * dx + dy2
                m = d2 < R2
                nm = plsc.all_reduce_population_count(m)[0]

                @pl.when(nm > 0)
                def _():
                    cz0 = jnp.where(m, z, INF)
                    cidx0 = jnp.full((16,), pid, jnp.int32)
                    og = occs[g]
                    nit = jnp.minimum(og + 1, K)

                    def ins(s, carry):
                        cz, cd, cidx = carry
                        off = s * S + base
                        sz = slz[pl.ds(off, 16)]
                        sd = sld[pl.ds(off, 16)]
                        si = sli[pl.ds(off, 16)]
                        take = cz < sz
                        slz[pl.ds(off, 16)] = jnp.where(take, cz, sz)
                        sld[pl.ds(off, 16)] = jnp.where(take, cd, sd)
                        sli[pl.ds(off, 16)] = jnp.where(take, cidx, si)
                        return (jnp.where(take, sz, cz),
                                jnp.where(take, sd, cd),
                                jnp.where(take, si, cidx))
                    lax.fori_loop(0, nit, ins, (cz0, d2, cidx0))
                    occs[g] = nit

            group(v0)

            @pl.when(v1 > v0)
            def _():
                group(v1)
            return 0
        lax.fori_loop(0, cnt, cand_body, 0)

        def comp_body(g, ec):
            base = g * 16
            pxcol = iota + base
            emptyv[pl.ds(base, 16)] = jnp.where(slz[pl.ds(base, 16)] < INF,
                                                0.0, 1.0)
            og = occs[g]

            def slot(s, carry):
                ec, tvec = carry
                off = s * S + base
                zs_ = slz[pl.ds(off, 16)]
                d2s = sld[pl.ds(off, 16)]
                pis = sli[pl.ds(off, 16)]
                valid = zs_ < INF
                dist = jnp.clip(d2s * INV_R2, 0.001, 1.0)
                a = jnp.maximum(1.0 - _sqrt16(dist), 0.0)
                a = jnp.where(valid, a, 0.0)
                w = a * tvec
                tvec = tvec * (1.0 - a)
                m = w > 0.0
                n = plsc.all_reduce_population_count(m)[0]
                plsc.store_compressed(ent_i.at[pl.ds(ec, 16)], pis, mask=m)
                plsc.store_compressed(ent_w.at[pl.ds(ec, 16)], w, mask=m)
                plsc.store_compressed(ent_p.at[pl.ds(ec, 16)], pxcol, mask=m)
                return (ec + n, tvec)
            ec, _unused = lax.fori_loop(
                0, og, slot, (ec, jnp.full((16,), jnp.float32(1.0))))
            return ec
        ec = lax.fori_loop(0, NV, comp_body, jnp.int32(0))

        ent_i[pl.ds(ec, 16)] = zeros16i
        ent_w[pl.ds(ec, 16)] = zeros16
        ent_p[pl.ds(ec, 16)] = zeros16i

        nch = lax.shift_right_logical(ec + 15, 4)
        nsup = lax.div(nch + (NWIN - 1), jnp.int32(NWIN))

        def sup_body(sp, _):
            ch0 = sp * NWIN
            nwc = jnp.minimum(nch - ch0, NWIN)

            def fire(i, _):
                pltpu.async_copy(
                    feats_h.at[ent_i.at[pl.ds((ch0 + i) * 16, 16)]],
                    fbuf.at[pl.ds(i * 16, 16)], sem)
                return 0
            lax.fori_loop(0, nwc, fire, 0)

            def drain(i, _):
                pltpu.make_async_copy(
                    feats_h.at[ent_i.at[pl.ds((ch0 + i) * 16, 16)]],
                    fbuf.at[pl.ds(i * 16, 16)], sem).wait()
                return 0
            lax.fori_loop(0, nwc, drain, 0)

            def acc(i, _):
                wv = ent_w[pl.ds((ch0 + i) * 16, 16)]
                pv = ent_p[pl.ds((ch0 + i) * 16, 16)]
                for e in range(16):
                    w_e = wv[e]
                    px_e = pv[e]
                    obase = px_e * C
                    fb = i * 16 + e
                    for q in range(C // 16):
                        plsc.addupdate(orow.at[pl.ds(obase + q * 16, 16)],
                                       w_e * fbuf[fb, pl.ds(q * 16, 16)])
                return 0
            lax.fori_loop(0, nwc, acc, 0)
            return 0
        lax.fori_loop(0, nsup, sup_body, 0)

        pltpu.sync_copy(orow, out_h.at[g_row])
        pltpu.sync_copy(emptyv, empty_h.at[g_row])
        return 0

    lax.fori_loop(0, ROWS_PER_W, row_body, 0)


def _sc_raster(xs, ys, zs, feats):
    mesh = plsc.VectorSubcoreMesh(core_axis_name="c", subcore_axis_name="s")
    f = pl.kernel(
        _raster_body,
        out_type=(
            jax.ShapeDtypeStruct((ROWS, S * C), jnp.float32),
            jax.ShapeDtypeStruct((ROWS, S), jnp.float32),
        ),
        mesh=mesh,
        compiler_params=pltpu.CompilerParams(needs_layout_passes=False,
                                             use_tc_tiling_on_sc=False),
        scratch_types=[
            pltpu.VMEM((B * P,), jnp.float32),
            pltpu.VMEM((B * P,), jnp.float32),
            pltpu.VMEM((B * P,), jnp.float32),
            pltpu.VMEM((S,), jnp.float32),
            pltpu.VMEM((K * S,), jnp.float32),
            pltpu.VMEM((K * S,), jnp.float32),
            pltpu.VMEM((K * S,), jnp.int32),
            pltpu.VMEM((P + 16,), jnp.int32),
            pltpu.VMEM((P + 16,), jnp.float32),
            pltpu.VMEM((P + 16,), jnp.float32),
            pltpu.VMEM((P + 16,), jnp.float32),
            pltpu.VMEM((ENT_CAP,), jnp.int32),
            pltpu.VMEM((ENT_CAP,), jnp.float32),
            pltpu.VMEM((ENT_CAP,), jnp.int32),
            pltpu.VMEM((S,), jnp.float32),
            pltpu.VMEM((NWIN * 16, C), jnp.float32),
            pltpu.VMEM((S * C,), jnp.float32),
            pltpu.SMEM((NV,), jnp.int32),
            pltpu.SemaphoreType.DMA,
        ],
    )
    return f(xs, ys, zs, feats)


def _bg_body(e_ref, m_ref, o_ref):
    e = e_ref[0]
    m = m_ref[...]
    t = jax.lax.dot_general(m, e, (((1,), (0,)), ((), ())),
                            preferred_element_type=jnp.float32)
    t2 = jax.lax.dot_general(t, m, (((1,), (0,)), ((), ())),
                             preferred_element_type=jnp.float32)
    o_ref[0] = (t2 > 0.0).astype(jnp.float32)


def _bg_dilate(empty):
    band = (jnp.abs(jnp.arange(S)[:, None] - jnp.arange(S)[None, :]) <= 1)
    m = band.astype(jnp.float32)
    return pl.pallas_call(
        _bg_body,
        out_shape=jax.ShapeDtypeStruct((B, S, S), jnp.float32),
        grid=(B,),
        in_specs=[pl.BlockSpec((1, S, S), lambda i: (i, 0, 0)),
                  pl.BlockSpec((S, S), lambda i: (0, 0))],
        out_specs=pl.BlockSpec((1, S, S), lambda i: (i, 0, 0)),
    )(empty, m)


def kernel(pts3D, src):
    xs = pts3D[..., 0].reshape(B * P)
    ys = pts3D[..., 1].reshape(B * P)
    zs = pts3D[..., 2].reshape(B * P)
    feats = jnp.transpose(src, (0, 2, 1)).reshape(B * P, C)
    out_hwc, empty = _sc_raster(xs, ys, zs, feats)
    out = jnp.transpose(out_hwc.reshape(B, S, S, C), (0, 3, 1, 2))
    bg = _bg_dilate(empty.reshape(B, S, S))
    return out, bg.astype(jnp.bool_)


# speedup vs baseline: 335.3639x; 1.0280x over previous
import functools

import jax
import jax.numpy as jnp
from jax import lax
from jax.experimental import pallas as pl
from jax.experimental.pallas import tpu as pltpu
from jax.experimental.pallas import tpu_sc as plsc

B = 2
P = 4096
C = 64
S = 256
K = 8
RADIUS = 1.5 / S * 2.0
R2 = RADIUS * RADIUS
INV_R2 = 1.0 / R2

NW = 32
ROWS = B * S
ROWS_PER_W = ROWS // NW
NV = S // 16
PV = P // 16
ENT_CAP = K * S + 16
NWIN = 48
INF = float("inf")


def _sqrt16(v):
    u = plsc.bitcast(v, jnp.int32)
    u = jnp.int32(0x5F3759DF) - lax.shift_right_logical(u, 1)
    y = plsc.bitcast(u, jnp.float32)
    for _ in range(3):
        y = y * (1.5 - 0.5 * v * y * y)
    return v * y


def _raster_body(xs_h, ys_h, zs_h, feats_h, out_h, empty_h,
                 xv, yv, zv, pxv, slz, sld, sli,
                 candp, candx, candz, candd,
                 ent_i, ent_w, ent_p, emptyv, fbuf, orow, occs, sem):
    cid = lax.axis_index("c")
    sid = lax.axis_index("s")
    wid = sid * 2 + cid

    pltpu.sync_copy(xs_h, xv)
    pltpu.sync_copy(ys_h, yv)
    pltpu.sync_copy(zs_h, zv)

    iota = lax.iota(jnp.int32, 16)
    iotaf = iota.astype(jnp.float32)

    def px_init(g, _):
        colf = iotaf + jnp.float32(g * 16)
        pxv[pl.ds(g * 16, 16)] = 1.0 - (2.0 * colf + 1.0) * jnp.float32(1.0 / S)
        return 0
    lax.fori_loop(0, NV, px_init, 0)

    zeros16 = jnp.zeros((16,), jnp.float32)
    zeros16i = jnp.zeros((16,), jnp.int32)
    inf16 = jnp.full((16,), INF, jnp.float32)

    def zorow(i, _):
        for u in range(4):
            orow[pl.ds(i * 64 + u * 16, 16)] = zeros16
        return 0
    lax.fori_loop(0, S * C // 64, zorow, 0)

    def row_body(t, _):
        g_row = wid + NW * t
        b = g_row // S
        j = g_row - b * S
        pyj = 1.0 - (2.0 * jnp.float32(j) + 1.0) * jnp.float32(1.0 / S)
        pbase = b * P

        def zslots(i, _):
            for u in range(4):
                slz[pl.ds(i * 64 + u * 16, 16)] = inf16
            return 0
        lax.fori_loop(0, K * NV // 4, zslots, 0)

        def zocc(i, _):
            occs[i] = jnp.int32(0)
            return 0
        lax.fori_loop(0, NV, zocc, 0)

        def scan_body(i, cnt):
            off = pbase + i * 16
            vy = yv[pl.ds(off, 16)]
            dy = pyj + vy
            dy2 = dy * dy
            m = dy2 < R2
            n = plsc.all_reduce_population_count(m)[0]

            @pl.when(n > 0)
            def _():
                pid = iota + off
                plsc.store_compressed(candp.at[pl.ds(cnt, 16)], pid, mask=m)
                plsc.store_compressed(candx.at[pl.ds(cnt, 16)], xv[pl.ds(off, 16)], mask=m)
                plsc.store_compressed(candz.at[pl.ds(cnt, 16)], zv[pl.ds(off, 16)], mask=m)
                plsc.store_compressed(candd.at[pl.ds(cnt, 16)], dy2, mask=m)
            return cnt + n
        cnt = lax.fori_loop(0, PV, scan_body, jnp.int32(0))

        def cand_body(ci, _):
            x = candx[pl.ds(ci, 16)][0]
            z = candz[pl.ds(ci, 16)][0]
            dy2 = candd[pl.ds(ci, 16)][0]
            pid = candp[pl.ds(ci, 16)][0]
            lo_f = jnp.clip((1.0 + x - RADIUS) * (S * 0.5) - 1.5, 0.0, S - 1.0)
            hi_f = jnp.clip((1.0 + x + RADIUS) * (S * 0.5) + 0.5, 0.0, S - 1.0)
            i_lo = lo_f.astype(jnp.int32)
            i_hi = hi_f.astype(jnp.int32)
            v0 = lax.shift_right_logical(i_lo, 4)
            v1 = lax.shift_right_logical(i_hi, 4)

            def group(g):
                base = g * 16
                pxg = pxv[pl.ds(base, 16)]
                dx = pxg + x
                d2 = dx * dx + dy2
                m = d2 < R2
                nm = plsc.all_reduce_population_count(m)[0]

                @pl.when(nm > 0)
                def _():
                    cz0 = jnp.where(m, z, INF)
                    cidx0 = jnp.full((16,), pid, jnp.int32)
                    og = occs[g]
                    nit = jnp.minimum(og + 1, K)

                    def ins(s, carry):
                        cz, cd, cidx = carry
                        off = s * S + base
                        sz = slz[pl.ds(off, 16)]
                        sd = sld[pl.ds(off, 16)]
                        si = sli[pl.ds(off, 16)]
                        take = cz < sz
                        slz[pl.ds(off, 16)] = jnp.where(take, cz, sz)
                        sld[pl.ds(off, 16)] = jnp.where(take, cd, sd)
                        sli[pl.ds(off, 16)] = jnp.where(take, cidx, si)
                        return (jnp.where(take, sz, cz),
                                jnp.where(take, sd, cd),
                                jnp.where(take, si, cidx))
                    lax.fori_loop(0, nit, ins, (cz0, d2, cidx0))
                    occs[g] = nit

            group(v0)

            @pl.when(v1 > v0)
            def _():
                group(v1)
            return 0
        lax.fori_loop(0, cnt, cand_body, 0)

        def comp_body(g, ec):
            base = g * 16
            pxcol = iota + base
            emptyv[pl.ds(base, 16)] = jnp.where(slz[pl.ds(base, 16)] < INF,
                                                0.0, 1.0)
            og = occs[g]

            def slot(s, carry):
                ec, tvec = carry
                off = s * S + base
                zs_ = slz[pl.ds(off, 16)]
                d2s = sld[pl.ds(off, 16)]
                pis = sli[pl.ds(off, 16)]
                valid = zs_ < INF
                dist = jnp.clip(d2s * INV_R2, 0.001, 1.0)
                a = jnp.maximum(1.0 - _sqrt16(dist), 0.0)
                a = jnp.where(valid, a, 0.0)
                w = a * tvec
                tvec = tvec * (1.0 - a)
                m = w > 0.0
                n = plsc.all_reduce_population_count(m)[0]
                plsc.store_compressed(ent_i.at[pl.ds(ec, 16)], pis, mask=m)
                plsc.store_compressed(ent_w.at[pl.ds(ec, 16)], w, mask=m)
                plsc.store_compressed(ent_p.at[pl.ds(ec, 16)], pxcol, mask=m)
                return (ec + n, tvec)
            ec, _unused = lax.fori_loop(
                0, og, slot, (ec, jnp.full((16,), jnp.float32(1.0))))
            return ec
        ec = lax.fori_loop(0, NV, comp_body, jnp.int32(0))

        ent_i[pl.ds(ec, 16)] = zeros16i
        ent_w[pl.ds(ec, 16)] = zeros16
        ent_p[pl.ds(ec, 16)] = zeros16i

        nch = lax.shift_right_logical(ec + 15, 4)
        nsup = lax.div(nch + (NWIN - 1), jnp.int32(NWIN))

        def sup_body(sp, _):
            ch0 = sp * NWIN
            nwc = jnp.minimum(nch - ch0, NWIN)

            def fire(i, _):
                pltpu.async_copy(
                    feats_h.at[ent_i.at[pl.ds((ch0 + i) * 16, 16)]],
                    fbuf.at[pl.ds(i * 16, 16)], sem)
                return 0
            lax.fori_loop(0, nwc, fire, 0)

            def drain(i, _):
                pltpu.make_async_copy(
                    feats_h.at[ent_i.at[pl.ds((ch0 + i) * 16, 16)]],
                    fbuf.at[pl.ds(i * 16, 16)], sem).wait()
                return 0
            lax.fori_loop(0, nwc, drain, 0)

            def acc(i, _):
                wv = ent_w[pl.ds((ch0 + i) * 16, 16)]
                pv = ent_p[pl.ds((ch0 + i) * 16, 16)]
                for e in range(16):
                    w_e = wv[e]
                    px_e = pv[e]
                    obase = px_e * C
                    fb = i * 16 + e
                    for q in range(C // 16):
                        plsc.addupdate(orow.at[pl.ds(obase + q * 16, 16)],
                                       w_e * fbuf[fb, pl.ds(q * 16, 16)])
                return 0
            lax.fori_loop(0, nwc, acc, 0)
            return 0
        lax.fori_loop(0, nsup, sup_body, 0)

        pltpu.sync_copy(orow, out_h.at[g_row])
        pltpu.sync_copy(emptyv, empty_h.at[g_row])

        def rez(i, _):
            pv = ent_p[pl.ds(i * 16, 16)]
            for e in range(16):
                obase = pv[e] * C
                for q in range(C // 16):
                    orow[pl.ds(obase + q * 16, 16)] = zeros16
            return 0
        lax.fori_loop(0, nch, rez, 0)
        return 0

    lax.fori_loop(0, ROWS_PER_W, row_body, 0)


def _sc_raster(xs, ys, zs, feats):
    mesh = plsc.VectorSubcoreMesh(core_axis_name="c", subcore_axis_name="s")
    f = pl.kernel(
        _raster_body,
        out_type=(
            jax.ShapeDtypeStruct((ROWS, S * C), jnp.float32),
            jax.ShapeDtypeStruct((ROWS, S), jnp.float32),
        ),
        mesh=mesh,
        compiler_params=pltpu.CompilerParams(needs_layout_passes=False,
                                             use_tc_tiling_on_sc=False),
        scratch_types=[
            pltpu.VMEM((B * P,), jnp.float32),
            pltpu.VMEM((B * P,), jnp.float32),
            pltpu.VMEM((B * P,), jnp.float32),
            pltpu.VMEM((S,), jnp.float32),
            pltpu.VMEM((K * S,), jnp.float32),
            pltpu.VMEM((K * S,), jnp.float32),
            pltpu.VMEM((K * S,), jnp.int32),
            pltpu.VMEM((P + 16,), jnp.int32),
            pltpu.VMEM((P + 16,), jnp.float32),
            pltpu.VMEM((P + 16,), jnp.float32),
            pltpu.VMEM((P + 16,), jnp.float32),
            pltpu.VMEM((ENT_CAP,), jnp.int32),
            pltpu.VMEM((ENT_CAP,), jnp.float32),
            pltpu.VMEM((ENT_CAP,), jnp.int32),
            pltpu.VMEM((S,), jnp.float32),
            pltpu.VMEM((NWIN * 16, C), jnp.float32),
            pltpu.VMEM((S * C,), jnp.float32),
            pltpu.SMEM((NV,), jnp.int32),
            pltpu.SemaphoreType.DMA,
        ],
    )
    return f(xs, ys, zs, feats)


def _bg_body(e_ref, m_ref, o_ref):
    e = e_ref[0]
    m = m_ref[...]
    t = jax.lax.dot_general(m, e, (((1,), (0,)), ((), ())),
                            preferred_element_type=jnp.float32)
    t2 = jax.lax.dot_general(t, m, (((1,), (0,)), ((), ())),
                             preferred_element_type=jnp.float32)
    o_ref[0] = (t2 > 0.0).astype(jnp.float32)


def _bg_dilate(empty):
    band = (jnp.abs(jnp.arange(S)[:, None] - jnp.arange(S)[None, :]) <= 1)
    m = band.astype(jnp.float32)
    return pl.pallas_call(
        _bg_body,
        out_shape=jax.ShapeDtypeStruct((B, S, S), jnp.float32),
        grid=(B,),
        in_specs=[pl.BlockSpec((1, S, S), lambda i: (i, 0, 0)),
                  pl.BlockSpec((S, S), lambda i: (0, 0))],
        out_specs=pl.BlockSpec((1, S, S), lambda i: (i, 0, 0)),
    )(empty, m)


def kernel(pts3D, src):
    xs = pts3D[..., 0].reshape(B * P)
    ys = pts3D[..., 1].reshape(B * P)
    zs = pts3D[..., 2].reshape(B * P)
    feats = jnp.transpose(src, (0, 2, 1)).reshape(B * P, C)
    out_hwc, empty = _sc_raster(xs, ys, zs, feats)
    out = jnp.transpose(out_hwc.reshape(B, S, S, C), (0, 3, 1, 2))
    bg = _bg_dilate(empty.reshape(B, S, S))
    return out, bg.astype(jnp.bool_)


# speedup vs baseline: 363.1996x; 1.0830x over previous
import functools

import jax
import jax.numpy as jnp
from jax import lax
from jax.experimental import pallas as pl
from jax.experimental.pallas import tpu as pltpu
from jax.experimental.pallas import tpu_sc as plsc

B = 2
P = 4096
C = 64
S = 256
K = 8
RADIUS = 1.5 / S * 2.0
R2 = RADIUS * RADIUS
INV_R2 = 1.0 / R2

NW = 32
ROWS = B * S
ROWS_PER_W = ROWS // NW
NV = S // 16
PV = P // 16
ENT_CAP = K * S + 16
NWIN = 48
INF = float("inf")


def _sqrt16(v):
    u = plsc.bitcast(v, jnp.int32)
    u = jnp.int32(0x5F3759DF) - lax.shift_right_logical(u, 1)
    y = plsc.bitcast(u, jnp.float32)
    for _ in range(3):
        y = y * (1.5 - 0.5 * v * y * y)
    return v * y


def _raster_body(xs_h, ys_h, zs_h, feats_h, out_h, empty_h,
                 xv, yv, zv, pxv, slz, sld, sli,
                 candp, hp, ht,
                 ent_i, ent_w, ent_p, emptyv, fbuf, orow, occs, sem):
    cid = lax.axis_index("c")
    sid = lax.axis_index("s")
    wid = sid * 2 + cid

    pltpu.sync_copy(xs_h, xv.at[pl.ds(0, B * P)])
    pltpu.sync_copy(ys_h, yv.at[pl.ds(0, B * P)])
    pltpu.sync_copy(zs_h, zv.at[pl.ds(0, B * P)])

    iota = lax.iota(jnp.int32, 16)
    iotaf = iota.astype(jnp.float32)

    def px_init(g, _):
        colf = iotaf + jnp.float32(g * 16)
        pxv[pl.ds(g * 16, 16)] = 1.0 - (2.0 * colf + 1.0) * jnp.float32(1.0 / S)
        return 0
    lax.fori_loop(0, NV, px_init, 0)

    zeros16 = jnp.zeros((16,), jnp.float32)
    zeros16i = jnp.zeros((16,), jnp.int32)
    inf16 = jnp.full((16,), INF, jnp.float32)

    def zorow(i, _):
        for u in range(4):
            orow[pl.ds(i * 64 + u * 16, 16)] = zeros16
        return 0
    lax.fori_loop(0, S * C // 64, zorow, 0)

    wid_v = jnp.full((16,), jnp.int32(0)) + wid
    hcnt = jnp.int32(0)
    for b in range(B):
        def pre_body(i, hc, b=b):
            off = b * P + i * 16
            vy = yv[pl.ds(off, 16)]
            qlo = (1.0 + vy - RADIUS) * (S * 0.5) - 1.5
            qhi = (1.0 + vy + RADIUS) * (S * 0.5) + 0.5
            jlo = (jnp.clip(qlo, -2.0, 256.0) + 256.0).astype(jnp.int32) - 256
            jhi = (jnp.clip(qhi, -2.0, 256.0) + 256.0).astype(jnp.int32) - 256
            jj = jlo + ((wid_v - jlo) & 31)
            hit = (jj <= jhi) & (jj >= 0) & (jj <= (S - 1))
            tt = lax.shift_right_arithmetic(jj - wid, 5) + (b * ROWS_PER_W // B)
            n = plsc.all_reduce_population_count(hit)[0]

            @pl.when(n > 0)
            def _():
                pid = iota + off
                plsc.store_compressed(hp.at[pl.ds(hc, 16)], pid, mask=hit)
                plsc.store_compressed(ht.at[pl.ds(hc, 16)], tt, mask=hit)
            return hc + n
        hcnt = lax.fori_loop(0, PV, pre_body, hcnt)
    ht[pl.ds(hcnt, 16)] = zeros16i - 1
    nhv = lax.shift_right_logical(hcnt + 15, 4)

    def row_body(t, _):
        g_row = wid + NW * t
        b = g_row // S
        j = g_row - b * S
        pyj = 1.0 - (2.0 * jnp.float32(j) + 1.0) * jnp.float32(1.0 / S)
        pbase = b * P

        def zslots(i, _):
            for u in range(4):
                slz[pl.ds(i * 64 + u * 16, 16)] = inf16
            return 0
        lax.fori_loop(0, K * NV // 4, zslots, 0)

        def zocc(i, _):
            occs[i] = jnp.int32(0)
            return 0
        lax.fori_loop(0, NV, zocc, 0)

        def scan_body(k, cnt):
            tv = ht[pl.ds(k * 16, 16)]
            m = tv == t
            n = plsc.all_reduce_population_count(m)[0]

            @pl.when(n > 0)
            def _():
                plsc.store_compressed(candp.at[pl.ds(cnt, 16)],
                                      hp[pl.ds(k * 16, 16)], mask=m)
            return cnt + n
        cnt = lax.fori_loop(0, nhv, scan_body, jnp.int32(0))

        def cand_body(ci, _):
            pid = candp[pl.ds(ci, 16)][0]
            x = xv[pl.ds(pid, 16)][0]
            z = zv[pl.ds(pid, 16)][0]
            dy = pyj + yv[pl.ds(pid, 16)][0]
            dy2 = dy * dy
            lo_f = jnp.clip((1.0 + x - RADIUS) * (S * 0.5) - 1.5, 0.0, S - 1.0)
            hi_f = jnp.clip((1.0 + x + RADIUS) * (S * 0.5) + 0.5, 0.0, S - 1.0)
            i_lo = lo_f.astype(jnp.int32)
            i_hi = hi_f.astype(jnp.int32)
            v0 = lax.shift_right_logical(i_lo, 4)
            v1 = lax.shift_right_logical(i_hi, 4)

            def group(g):
                base = g * 16
                pxg = pxv[pl.ds(base, 16)]
                dx = pxg + x
                d2 = dx * dx + dy2
                m = d2 < R2
                nm = plsc.all_reduce_population_count(m)[0]

                @pl.when(nm > 0)
                def _():
                    cz0 = jnp.where(m, z, INF)
                    cidx0 = jnp.full((16,), pid, jnp.int32)
                    og = occs[g]
                    nit = jnp.minimum(og + 1, K)

                    def ins(s, carry):
                        cz, cd, cidx = carry
                        off = s * S + base
                        sz = slz[pl.ds(off, 16)]
                        sd = sld[pl.ds(off, 16)]
                        si = sli[pl.ds(off, 16)]
                        take = cz < sz
                        slz[pl.ds(off, 16)] = jnp.where(take, cz, sz)
                        sld[pl.ds(off, 16)] = jnp.where(take, cd, sd)
                        sli[pl.ds(off, 16)] = jnp.where(take, cidx, si)
                        return (jnp.where(take, sz, cz),
                                jnp.where(take, sd, cd),
                                jnp.where(take, si, cidx))
                    lax.fori_loop(0, nit, ins, (cz0, d2, cidx0))
                    occs[g] = nit

            group(v0)

            @pl.when(v1 > v0)
            def _():
                group(v1)
            return 0
        lax.fori_loop(0, cnt, cand_body, 0)

        def comp_body(g, ec):
            base = g * 16
            pxcol = iota + base
            emptyv[pl.ds(base, 16)] = jnp.where(slz[pl.ds(base, 16)] < INF,
                                                0.0, 1.0)
            og = occs[g]

            def slot(s, carry):
                ec, tvec = carry
                off = s * S + base
                zs_ = slz[pl.ds(off, 16)]
                d2s = sld[pl.ds(off, 16)]
                pis = sli[pl.ds(off, 16)]
                valid = zs_ < INF
                dist = jnp.clip(d2s * INV_R2, 0.001, 1.0)
                a = jnp.maximum(1.0 - _sqrt16(dist), 0.0)
                a = jnp.where(valid, a, 0.0)
                w = a * tvec
                tvec = tvec * (1.0 - a)
                m = w > 0.0
                n = plsc.all_reduce_population_count(m)[0]
                plsc.store_compressed(ent_i.at[pl.ds(ec, 16)], pis, mask=m)
                plsc.store_compressed(ent_w.at[pl.ds(ec, 16)], w, mask=m)
                plsc.store_compressed(ent_p.at[pl.ds(ec, 16)], pxcol, mask=m)
                return (ec + n, tvec)
            ec, _unused = lax.fori_loop(
                0, og, slot, (ec, jnp.full((16,), jnp.float32(1.0))))
            return ec
        ec = lax.fori_loop(0, NV, comp_body, jnp.int32(0))

        ent_i[pl.ds(ec, 16)] = zeros16i
        ent_w[pl.ds(ec, 16)] = zeros16
        ent_p[pl.ds(ec, 16)] = zeros16i

        nch = lax.shift_right_logical(ec + 15, 4)
        nsup = lax.div(nch + (NWIN - 1), jnp.int32(NWIN))

        def sup_body(sp, _):
            ch0 = sp * NWIN
            nwc = jnp.minimum(nch - ch0, NWIN)

            def fire(i, _):
                pltpu.async_copy(
                    feats_h.at[ent_i.at[pl.ds((ch0 + i) * 16, 16)]],
                    fbuf.at[pl.ds(i * 16, 16)], sem)
                return 0
            lax.fori_loop(0, nwc, fire, 0)

            def drain(i, _):
                pltpu.make_async_copy(
                    feats_h.at[ent_i.at[pl.ds((ch0 + i) * 16, 16)]],
                    fbuf.at[pl.ds(i * 16, 16)], sem).wait()
                return 0
            lax.fori_loop(0, nwc, drain, 0)

            def acc(i, _):
                wv = ent_w[pl.ds((ch0 + i) * 16, 16)]
                pv = ent_p[pl.ds((ch0 + i) * 16, 16)]
                for e in range(16):
                    w_e = wv[e]
                    px_e = pv[e]
                    obase = px_e * C
                    fb = i * 16 + e
                    for q in range(C // 16):
                        plsc.addupdate(orow.at[pl.ds(obase + q * 16, 16)],
                                       w_e * fbuf[fb, pl.ds(q * 16, 16)])
                return 0
            lax.fori_loop(0, nwc, acc, 0)
            return 0
        lax.fori_loop(0, nsup, sup_body, 0)

        pltpu.sync_copy(orow, out_h.at[g_row])
        pltpu.sync_copy(emptyv, empty_h.at[g_row])

        def rez(i, _):
            pv = ent_p[pl.ds(i * 16, 16)]
            for e in range(16):
                obase = pv[e] * C
                for q in range(C // 16):
                    orow[pl.ds(obase + q * 16, 16)] = zeros16
            return 0
        lax.fori_loop(0, nch, rez, 0)
        return 0

    lax.fori_loop(0, ROWS_PER_W, row_body, 0)


def _sc_raster(xs, ys, zs, feats):
    mesh = plsc.VectorSubcoreMesh(core_axis_name="c", subcore_axis_name="s")
    f = pl.kernel(
        _raster_body,
        out_type=(
            jax.ShapeDtypeStruct((ROWS, S * C), jnp.float32),
            jax.ShapeDtypeStruct((ROWS, S), jnp.float32),
        ),
        mesh=mesh,
        compiler_params=pltpu.CompilerParams(needs_layout_passes=False,
                                             use_tc_tiling_on_sc=False),
        scratch_types=[
            pltpu.VMEM((B * P + 16,), jnp.float32),
            pltpu.VMEM((B * P + 16,), jnp.float32),
            pltpu.VMEM((B * P + 16,), jnp.float32),
            pltpu.VMEM((S,), jnp.float32),
            pltpu.VMEM((K * S,), jnp.float32),
            pltpu.VMEM((K * S,), jnp.float32),
            pltpu.VMEM((K * S,), jnp.int32),
            pltpu.VMEM((P + 16,), jnp.int32),
            pltpu.VMEM((B * P + 16,), jnp.int32),
            pltpu.VMEM((B * P + 16,), jnp.int32),
            pltpu.VMEM((ENT_CAP,), jnp.int32),
            pltpu.VMEM((ENT_CAP,), jnp.float32),
            pltpu.VMEM((ENT_CAP,), jnp.int32),
            pltpu.VMEM((S,), jnp.float32),
            pltpu.VMEM((NWIN * 16, C), jnp.float32),
            pltpu.VMEM((S * C,), jnp.float32),
            pltpu.SMEM((NV,), jnp.int32),
            pltpu.SemaphoreType.DMA,
        ],
    )
    return f(xs, ys, zs, feats)


def _bg_body(e_ref, m_ref, o_ref):
    e = e_ref[0]
    m = m_ref[...]
    t = jax.lax.dot_general(m, e, (((1,), (0,)), ((), ())),
                            preferred_element_type=jnp.float32)
    t2 = jax.lax.dot_general(t, m, (((1,), (0,)), ((), ())),
                             preferred_element_type=jnp.float32)
    o_ref[0] = (t2 > 0.0).astype(jnp.float32)


def _bg_dilate(empty):
    band = (jnp.abs(jnp.arange(S)[:, None] - jnp.arange(S)[None, :]) <= 1)
    m = band.astype(jnp.float32)
    return pl.pallas_call(
        _bg_body,
        out_shape=jax.ShapeDtypeStruct((B, S, S), jnp.float32),
        grid=(B,),
        in_specs=[pl.BlockSpec((1, S, S), lambda i: (i, 0, 0)),
                  pl.BlockSpec((S, S), lambda i: (0, 0))],
        out_specs=pl.BlockSpec((1, S, S), lambda i: (i, 0, 0)),
    )(empty, m)


def kernel(pts3D, src):
    xs = pts3D[..., 0].reshape(B * P)
    ys = pts3D[..., 1].reshape(B * P)
    zs = pts3D[..., 2].reshape(B * P)
    feats = jnp.transpose(src, (0, 2, 1)).reshape(B * P, C)
    out_hwc, empty = _sc_raster(xs, ys, zs, feats)
    out = jnp.transpose(out_hwc.reshape(B, S, S, C), (0, 3, 1, 2))
    bg = _bg_dilate(empty.reshape(B, S, S))
    return out, bg.astype(jnp.bool_)
